# baseline (speedup 1.0000x reference)
import functools

import jax
import jax.numpy as jnp
from jax import lax
from jax.experimental import pallas as pl
from jax.experimental.pallas import tpu as pltpu
from jax.experimental.pallas import tpu_sc as plsc

N = 10000
E = 320000
D = 128

NC = 2
NS = 16
NW = NC * NS
EPT = E // NW
K = 80
NCHUNK = EPT // K
RPT = 624
ZR = 16

_mesh = plsc.VectorSubcoreMesh(core_axis_name="c", subcore_axis_name="s")


@functools.partial(
    pl.kernel,
    out_type=jax.ShapeDtypeStruct((NC, N, D), jnp.float32),
    mesh=_mesh,
    scratch_types=[
        pltpu.VMEM((K,), jnp.int32),
        pltpu.VMEM((K,), jnp.int32),
        pltpu.VMEM((K,), jnp.int32),
        pltpu.VMEM((K,), jnp.int32),
        pltpu.VMEM((K,), jnp.int32),
        pltpu.VMEM((K,), jnp.int32),
        pltpu.VMEM((K,), jnp.int32),
        pltpu.VMEM((K,), jnp.int32),
        pltpu.VMEM((K, D), jnp.float32),
        pltpu.VMEM((K, D), jnp.float32),
        pltpu.VMEM((K, D), jnp.float32),
        pltpu.VMEM((K, D), jnp.float32),
        pltpu.VMEM((ZR, D), jnp.float32),
        pltpu.VMEM_SHARED((N, D), jnp.float32),
        pltpu.SemaphoreType.DMA,
        pltpu.SemaphoreType.DMA,
        pltpu.SemaphoreType.DMA,
        pltpu.SemaphoreType.DMA,
        pltpu.SemaphoreType.DMA,
        pltpu.SemaphoreType.DMA,
        pltpu.SemaphoreType.DMA,
        pltpu.SemaphoreType.DMA,
        pltpu.SemaphoreType.DMA,
        pltpu.SemaphoreType.DMA,
        pltpu.SemaphoreType.DMA,
        pltpu.SemaphoreType.DMA,
        pltpu.SemaphoreType.DMA,
        pltpu.SemaphoreType.DMA,
        pltpu.SemaphoreType.DMA,
        pltpu.SemaphoreType.DMA,
    ],
)
def _aggregate(x_hbm, edge_hbm, out_hbm,
               sv0, sv1, sv2, sv3, dv0, dv1, dv2, dv3,
               rb0, rb1, rb2, rb3, zbuf_v, acc_sh,
               gs0, gs1, gs2, gs3, ss0, ss1, ss2, ss3,
               is0, is1, is2, is3, id0, id1, id2, id3):
    srcv = [sv0, sv1, sv2, sv3]
    dstv = [dv0, dv1, dv2, dv3]
    rows = [rb0, rb1, rb2, rb3]
    gsem = [gs0, gs1, gs2, gs3]
    ssem = [ss0, ss1, ss2, ss3]
    isem = [is0, is1, is2, is3]
    dsem = [id0, id1, id2, id3]
    c = lax.axis_index("c")
    s = lax.axis_index("s")
    wid = s * NC + c
    ebase = wid * EPT

    zero = jnp.zeros((16,), jnp.float32)
    nlane = D // 16

    def zfill(i, carry):
        zbuf_v[i // nlane, pl.ds((i % nlane) * 16, 16)] = zero
        return carry

    lax.fori_loop(0, ZR * nlane, zfill, 0)

    def zslice(j, carry):
        pltpu.sync_copy(zbuf_v, acc_sh.at[pl.ds(s * RPT + j * ZR, ZR)])
        return carry

    lax.fori_loop(0, 640 // ZR, zslice, 0)

    plsc.subcore_barrier()

    def issue_sidx(chunk, b):
        pltpu.async_copy(edge_hbm.at[pl.ds(ebase + chunk * K, K)],
                         srcv[b], isem[b])

    def wait_sidx(b):
        pltpu.make_async_copy(edge_hbm.at[pl.ds(0, K)], srcv[b], isem[b]).wait()

    def issue_dst(chunk, b):
        pltpu.async_copy(edge_hbm.at[pl.ds(E + ebase + chunk * K, K)],
                         dstv[b], dsem[b])

    def wait_dst(b):
        pltpu.make_async_copy(edge_hbm.at[pl.ds(0, K)], dstv[b], dsem[b]).wait()

    def issue_gather(b):
        pltpu.async_copy(x_hbm.at[srcv[b]], rows[b], gsem[b])

    def wait_gather(b):
        pltpu.make_async_copy(x_hbm.at[pl.ds(0, K)], rows[b], gsem[b]).wait()

    def issue_scatter(b):
        pltpu.async_copy(rows[b], acc_sh.at[dstv[b]], ssem[b], add=True)

    def wait_scatter(b):
        pltpu.make_async_copy(rows[b], acc_sh.at[dstv[b]], ssem[b]).wait()

    def step(chunk, b, wait_s=True, issue_g=True, issue_d=True, issue_si=True):
        if wait_s:
            wait_scatter((b + 2) % 4)
        if issue_g:
            wait_sidx((b + 2) % 4)
            issue_gather((b + 2) % 4)
            if issue_d:
                issue_dst(chunk + 2, (b + 2) % 4)
        wait_gather(b)
        if issue_si:
            issue_sidx(chunk + 4, b)
        wait_dst(b)
        issue_scatter(b)

    for t in range(4):
        issue_sidx(t, t)
    issue_dst(0, 0)
    issue_dst(1, 1)
    issue_dst(2, 2)
    issue_dst(3, 3)
    wait_sidx(0)
    issue_gather(0)
    wait_sidx(1)
    issue_gather(1)

    def step01(chunk, b):
        wait_sidx((b + 2) % 4)
        issue_gather((b + 2) % 4)
        wait_gather(b)
        issue_sidx(chunk + 4, b)
        wait_dst(b)
        issue_scatter(b)

    step01(0, 0)
    step01(1, 1)

    NG = (NCHUNK - 5) // 4

    def group(j, carry):
        c0 = 4 * j + 2
        step(c0 + 0, 2)
        step(c0 + 1, 3)
        step(c0 + 2, 0)
        step(c0 + 3, 1)
        return carry

    lax.fori_loop(0, NG, group, 0)

    step(NCHUNK - 3, 2, issue_si=False)
    step(NCHUNK - 2, 3, issue_g=False, issue_si=False)
    step(NCHUNK - 1, 0, issue_g=False, issue_si=False)
    wait_scatter(3)
    wait_scatter(0)
    wait_sidx(1)

    plsc.subcore_barrier()

    pltpu.sync_copy(
        acc_sh.at[pl.ds(s * RPT, RPT)],
        out_hbm.at[c, pl.ds(s * RPT, RPT)],
    )

    @pl.when(s == NS - 1)
    def _tail():
        pltpu.sync_copy(
            acc_sh.at[pl.ds(NS * RPT, N - NS * RPT)],
            out_hbm.at[c, pl.ds(NS * RPT, N - NS * RPT)],
        )


BR = 1000


def _mm_body(p_ref, w_ref, o_ref):
    a = p_ref[0] + p_ref[1]
    o_ref[...] = jnp.dot(a, w_ref[...], preferred_element_type=jnp.float32)


def _matmul(partials, W):
    return pl.pallas_call(
        _mm_body,
        grid=(N // BR,),
        in_specs=[
            pl.BlockSpec((NC, BR, D), lambda i: (0, i, 0)),
            pl.BlockSpec((D, D), lambda i: (0, 0)),
        ],
        out_specs=pl.BlockSpec((BR, D), lambda i: (i, 0)),
        out_shape=jax.ShapeDtypeStruct((N, D), jnp.float32),
    )(partials, W)


@jax.jit
def _run(x, edge_index, W):
    partials = _aggregate(x, edge_index.astype(jnp.int32).reshape(-1))
    return _matmul(partials, W)


def kernel(x, edge_index, W):
    return _run(x, edge_index, W)


# speedup vs baseline: 1.0417x; 1.0417x over previous
import functools

import jax
import jax.numpy as jnp
from jax import lax
from jax.experimental import pallas as pl
from jax.experimental.pallas import tpu as pltpu
from jax.experimental.pallas import tpu_sc as plsc

N = 10000
E = 320000
D = 128

NC = 2
NS = 16
NW = NC * NS
EPT = E // NW
K = 80
NCHUNK = EPT // K
RPT = 624
ZR = 16

_mesh = plsc.VectorSubcoreMesh(core_axis_name="c", subcore_axis_name="s")


@functools.partial(
    pl.kernel,
    out_type=jax.ShapeDtypeStruct((NC, N, D), jnp.float32),
    mesh=_mesh,
    scratch_types=[
        pltpu.VMEM((EPT,), jnp.int32),
        pltpu.VMEM((K,), jnp.int32),
        pltpu.VMEM((K,), jnp.int32),
        pltpu.VMEM((K,), jnp.int32),
        pltpu.VMEM((K, D), jnp.float32),
        pltpu.VMEM((K, D), jnp.float32),
        pltpu.VMEM((K, D), jnp.float32),
        pltpu.VMEM((ZR, D), jnp.float32),
        pltpu.VMEM_SHARED((N, D), jnp.float32),
        pltpu.SemaphoreType.DMA,
        pltpu.SemaphoreType.DMA,
        pltpu.SemaphoreType.DMA,
        pltpu.SemaphoreType.DMA,
        pltpu.SemaphoreType.DMA,
        pltpu.SemaphoreType.DMA,
    ],
)
def _aggregate(x_hbm, edge_hbm, out_hbm, src_all,
               dv0, dv1, dv2, rb0, rb1, rb2,
               zbuf_v, acc_sh, gs0, gs1, gs2, ds0, ds1, ds2):
    dstv = [dv0, dv1, dv2]
    rows = [rb0, rb1, rb2]
    gsem = [gs0, gs1, gs2]
    dsem = [ds0, ds1, ds2]
    c = lax.axis_index("c")
    s = lax.axis_index("s")
    wid = s * NC + c
    ebase = wid * EPT

    pltpu.sync_copy(edge_hbm.at[pl.ds(ebase, EPT)], src_all)

    zero = jnp.zeros((16,), jnp.float32)
    nlane = D // 16

    def zfill(i, carry):
        zbuf_v[i // nlane, pl.ds((i % nlane) * 16, 16)] = zero
        return carry

    lax.fori_loop(0, ZR * nlane, zfill, 0)

    def zslice(j, carry):
        pltpu.sync_copy(zbuf_v, acc_sh.at[pl.ds(s * RPT + j * ZR, ZR)])
        return carry

    lax.fori_loop(0, 640 // ZR, zslice, 0)

    plsc.subcore_barrier()

    def issue_gather(chunk, b):
        pltpu.async_copy(x_hbm.at[src_all.at[pl.ds(chunk * K, K)]],
                         rows[b], gsem[b])

    def wait_gather(b):
        pltpu.make_async_copy(x_hbm.at[pl.ds(0, K)], rows[b], gsem[b]).wait()

    def issue_dst(chunk, b):
        pltpu.async_copy(edge_hbm.at[pl.ds(E + ebase + chunk * K, K)],
                         dstv[b], dsem[b])

    def wait_dst(b):
        pltpu.make_async_copy(edge_hbm.at[pl.ds(0, K)], dstv[b], dsem[b]).wait()

    def scatter(b):
        pltpu.sync_copy(rows[b], acc_sh.at[dstv[b]], add=True)

    def step(chunk, b, issue_g=True, issue_d=True):
        if issue_g:
            issue_gather(chunk + 2, (b + 2) % 3)
        if issue_d:
            issue_dst(chunk + 1, (b + 1) % 3)
        wait_gather(b)
        wait_dst(b)
        scatter(b)

    issue_gather(0, 0)
    issue_dst(0, 0)
    issue_gather(1, 1)
    step(0, 0)

    NG = (NCHUNK - 5) // 3

    def group(j, carry):
        c0 = 3 * j + 1
        step(c0 + 0, 1)
        step(c0 + 1, 2)
        step(c0 + 2, 0)
        return carry

    lax.fori_loop(0, NG, group, 0)

    step(NCHUNK - 4, (NCHUNK - 4) % 3)
    step(NCHUNK - 3, (NCHUNK - 3) % 3)
    step(NCHUNK - 2, (NCHUNK - 2) % 3, issue_g=False)
    step(NCHUNK - 1, (NCHUNK - 1) % 3, issue_g=False, issue_d=False)

    plsc.subcore_barrier()

    pltpu.sync_copy(
        acc_sh.at[pl.ds(s * RPT, RPT)],
        out_hbm.at[c, pl.ds(s * RPT, RPT)],
    )

    @pl.when(s == NS - 1)
    def _tail():
        pltpu.sync_copy(
            acc_sh.at[pl.ds(NS * RPT, N - NS * RPT)],
            out_hbm.at[c, pl.ds(NS * RPT, N - NS * RPT)],
        )


BR = 1000


def _mm_body(p_ref, w_ref, o_ref):
    a = p_ref[0] + p_ref[1]
    o_ref[...] = jnp.dot(a, w_ref[...], preferred_element_type=jnp.float32)


def _matmul(partials, W):
    return pl.pallas_call(
        _mm_body,
        grid=(N // BR,),
        in_specs=[
            pl.BlockSpec((NC, BR, D), lambda i: (0, i, 0)),
            pl.BlockSpec((D, D), lambda i: (0, 0)),
        ],
        out_specs=pl.BlockSpec((BR, D), lambda i: (i, 0)),
        out_shape=jax.ShapeDtypeStruct((N, D), jnp.float32),
    )(partials, W)


@jax.jit
def _run(x, edge_index, W):
    partials = _aggregate(x, edge_index.astype(jnp.int32).reshape(-1))
    return _matmul(partials, W)


def kernel(x, edge_index, W):
    return _run(x, edge_index, W)


# speedup vs baseline: 1.0899x; 1.0463x over previous
import functools

import jax
import jax.numpy as jnp
from jax import lax
from jax.experimental import pallas as pl
from jax.experimental.pallas import tpu as pltpu
from jax.experimental.pallas import tpu_sc as plsc

N = 10000
E = 320000
D = 128

NC = 2
NS = 16
NW = NC * NS
EPT = E // NW
K = 80
NCHUNK = EPT // K
RPT = 624
ZR = 16

_mesh = plsc.VectorSubcoreMesh(core_axis_name="c", subcore_axis_name="s")


@functools.partial(
    pl.kernel,
    out_type=jax.ShapeDtypeStruct((NC, N, D), jnp.float32),
    mesh=_mesh,
    scratch_types=[
        pltpu.VMEM((EPT,), jnp.int32),
        pltpu.VMEM((K,), jnp.int32),
        pltpu.VMEM((K,), jnp.int32),
        pltpu.VMEM((K,), jnp.int32),
        pltpu.VMEM((K, D), jnp.float32),
        pltpu.VMEM((K, D), jnp.float32),
        pltpu.VMEM((K, D), jnp.float32),
        pltpu.VMEM((ZR, D), jnp.float32),
        pltpu.VMEM_SHARED((N, D), jnp.float32),
        pltpu.SemaphoreType.DMA,
        pltpu.SemaphoreType.DMA,
        pltpu.SemaphoreType.DMA,
        pltpu.SemaphoreType.DMA,
        pltpu.SemaphoreType.DMA,
        pltpu.SemaphoreType.DMA,
        pltpu.SemaphoreType.DMA,
    ],
)
def _aggregate(x_hbm, edge_hbm, out_hbm, src_all,
               dv0, dv1, dv2, rb0, rb1, rb2,
               zbuf_v, acc_sh, gs0, gs1, gs2, ds0, ds1, ds2, psem):
    dstv = [dv0, dv1, dv2]
    rows = [rb0, rb1, rb2]
    gsem = [gs0, gs1, gs2]
    dsem = [ds0, ds1, ds2]
    c = lax.axis_index("c")
    s = lax.axis_index("s")
    wid = s * NC + c
    ebase = wid * EPT

    pltpu.async_copy(edge_hbm.at[pl.ds(ebase, EPT)], src_all, gsem[0])

    zero = jnp.zeros((16,), jnp.float32)
    nlane = D // 16

    def zfill(i, carry):
        zbuf_v[i // nlane, pl.ds((i % nlane) * 16, 16)] = zero
        return carry

    lax.fori_loop(0, ZR * nlane, zfill, 0)

    NZ = 640 // ZR

    def zissue(j, carry):
        pltpu.async_copy(zbuf_v, acc_sh.at[pl.ds(s * RPT + j * ZR, ZR)], psem)
        return carry

    lax.fori_loop(0, NZ, zissue, 0)

    def zdrain(j, carry):
        pltpu.make_async_copy(zbuf_v, acc_sh.at[pl.ds(0, ZR)], psem).wait()
        return carry

    lax.fori_loop(0, NZ, zdrain, 0)

    pltpu.make_async_copy(edge_hbm.at[pl.ds(0, EPT)], src_all, gsem[0]).wait()

    plsc.subcore_barrier()

    def issue_gather(chunk, b):
        pltpu.async_copy(x_hbm.at[src_all.at[pl.ds(chunk * K, K)]],
                         rows[b], gsem[b])

    def wait_gather(b):
        pltpu.make_async_copy(x_hbm.at[pl.ds(0, K)], rows[b], gsem[b]).wait()

    def issue_dst(chunk, b):
        pltpu.async_copy(edge_hbm.at[pl.ds(E + ebase + chunk * K, K)],
                         dstv[b], dsem[b])

    def wait_dst(b):
        pltpu.make_async_copy(edge_hbm.at[pl.ds(0, K)], dstv[b], dsem[b]).wait()

    def scatter(b):
        pltpu.sync_copy(rows[b], acc_sh.at[dstv[b]], add=True)

    def step(chunk, b, issue_g=True, issue_d=True):
        if issue_g:
            issue_gather(chunk + 2, (b + 2) % 3)
        if issue_d:
            issue_dst(chunk + 1, (b + 1) % 3)
        wait_gather(b)
        wait_dst(b)
        scatter(b)

    issue_gather(0, 0)
    issue_dst(0, 0)
    issue_gather(1, 1)
    step(0, 0)

    NG = (NCHUNK - 5) // 3

    def group(j, carry):
        c0 = 3 * j + 1
        step(c0 + 0, 1)
        step(c0 + 1, 2)
        step(c0 + 2, 0)
        return carry

    lax.fori_loop(0, NG, group, 0)

    step(NCHUNK - 4, (NCHUNK - 4) % 3)
    step(NCHUNK - 3, (NCHUNK - 3) % 3)
    step(NCHUNK - 2, (NCHUNK - 2) % 3, issue_g=False)
    step(NCHUNK - 1, (NCHUNK - 1) % 3, issue_g=False, issue_d=False)

    plsc.subcore_barrier()

    pltpu.sync_copy(
        acc_sh.at[pl.ds(s * RPT, RPT)],
        out_hbm.at[c, pl.ds(s * RPT, RPT)],
    )

    @pl.when(s == NS - 1)
    def _tail():
        pltpu.sync_copy(
            acc_sh.at[pl.ds(NS * RPT, N - NS * RPT)],
            out_hbm.at[c, pl.ds(NS * RPT, N - NS * RPT)],
        )


BR = 2000


def _mm_body(p_ref, w_ref, o_ref):
    a = p_ref[0] + p_ref[1]
    o_ref[...] = jnp.dot(a, w_ref[...], preferred_element_type=jnp.float32)


def _matmul(partials, W):
    return pl.pallas_call(
        _mm_body,
        grid=(N // BR,),
        in_specs=[
            pl.BlockSpec((NC, BR, D), lambda i: (0, i, 0)),
            pl.BlockSpec((D, D), lambda i: (0, 0)),
        ],
        out_specs=pl.BlockSpec((BR, D), lambda i: (i, 0)),
        out_shape=jax.ShapeDtypeStruct((N, D), jnp.float32),
    )(partials, W)


@jax.jit
def _run(x, edge_index, W):
    partials = _aggregate(x, edge_index.astype(jnp.int32).reshape(-1))
    return _matmul(partials, W)


def kernel(x, edge_index, W):
    return _run(x, edge_index, W)


# speedup vs baseline: 1.1040x; 1.0129x over previous
import functools

import jax
import jax.numpy as jnp
from jax import lax
from jax.experimental import pallas as pl
from jax.experimental.pallas import tpu as pltpu
from jax.experimental.pallas import tpu_sc as plsc

N = 10000
E = 320000
D = 128

NC = 2
NS = 16
NW = NC * NS
EPT = E // NW
K = 80
NCHUNK = EPT // K
RPT = 624
ZR = 16

_mesh = plsc.VectorSubcoreMesh(core_axis_name="c", subcore_axis_name="s")


@functools.partial(
    pl.kernel,
    out_type=jax.ShapeDtypeStruct((NC, N, D), jnp.float32),
    mesh=_mesh,
    scratch_types=[
        pltpu.VMEM((EPT,), jnp.int32),
        pltpu.VMEM((K,), jnp.int32),
        pltpu.VMEM((K,), jnp.int32),
        pltpu.VMEM((K,), jnp.int32),
        pltpu.VMEM((K, D), jnp.float32),
        pltpu.VMEM((K, D), jnp.float32),
        pltpu.VMEM((K, D), jnp.float32),
        pltpu.VMEM((ZR, D), jnp.float32),
        pltpu.VMEM_SHARED((N, D), jnp.float32),
        pltpu.SemaphoreType.DMA,
        pltpu.SemaphoreType.DMA,
        pltpu.SemaphoreType.DMA,
        pltpu.SemaphoreType.DMA,
        pltpu.SemaphoreType.DMA,
        pltpu.SemaphoreType.DMA,
        pltpu.SemaphoreType.DMA,
    ],
)
def _aggregate(x_hbm, edge_hbm, out_hbm, src_all,
               dv0, dv1, dv2, rb0, rb1, rb2,
               zbuf_v, acc_sh, gs0, gs1, gs2, ds0, ds1, ds2, psem):
    dstv = [dv0, dv1, dv2]
    rows = [rb0, rb1, rb2]
    gsem = [gs0, gs1, gs2]
    dsem = [ds0, ds1, ds2]
    c = lax.axis_index("c")
    s = lax.axis_index("s")
    wid = s * NC + c
    ebase = wid * EPT

    pltpu.async_copy(edge_hbm.at[pl.ds(ebase, EPT)], src_all, gsem[0])

    zero = jnp.zeros((16,), jnp.float32)
    nlane = D // 16

    def zfill(i, carry):
        zbuf_v[i // nlane, pl.ds((i % nlane) * 16, 16)] = zero
        return carry

    lax.fori_loop(0, ZR * nlane, zfill, 0)

    NZ = 640 // ZR

    def zissue(j, carry):
        pltpu.async_copy(zbuf_v, acc_sh.at[pl.ds(s * RPT + j * ZR, ZR)], psem)
        return carry

    lax.fori_loop(0, NZ, zissue, 0)

    pltpu.make_async_copy(edge_hbm.at[pl.ds(0, EPT)], src_all, gsem[0]).wait()

    def issue_gather(chunk, b):
        pltpu.async_copy(x_hbm.at[src_all.at[pl.ds(chunk * K, K)]],
                         rows[b], gsem[b])

    def wait_gather(b):
        pltpu.make_async_copy(x_hbm.at[pl.ds(0, K)], rows[b], gsem[b]).wait()

    def issue_dst(chunk, b):
        pltpu.async_copy(edge_hbm.at[pl.ds(E + ebase + chunk * K, K)],
                         dstv[b], dsem[b])

    def wait_dst(b):
        pltpu.make_async_copy(edge_hbm.at[pl.ds(0, K)], dstv[b], dsem[b]).wait()

    def scatter(b):
        pltpu.sync_copy(rows[b], acc_sh.at[dstv[b]], add=True)

    def step(chunk, b, issue_g=True, issue_d=True):
        if issue_g:
            issue_gather(chunk + 2, (b + 2) % 3)
        if issue_d:
            issue_dst(chunk + 1, (b + 1) % 3)
        wait_gather(b)
        wait_dst(b)
        scatter(b)

    issue_gather(0, 0)
    issue_dst(0, 0)
    issue_gather(1, 1)
    issue_dst(1, 1)

    def zdrain(j, carry):
        pltpu.make_async_copy(zbuf_v, acc_sh.at[pl.ds(0, ZR)], psem).wait()
        return carry

    lax.fori_loop(0, NZ, zdrain, 0)

    plsc.subcore_barrier()

    step(0, 0, issue_d=False)

    NG = (NCHUNK - 5) // 3

    def group(j, carry):
        c0 = 3 * j + 1
        step(c0 + 0, 1)
        step(c0 + 1, 2)
        step(c0 + 2, 0)
        return carry

    lax.fori_loop(0, NG, group, 0)

    step(NCHUNK - 4, (NCHUNK - 4) % 3)
    step(NCHUNK - 3, (NCHUNK - 3) % 3)
    step(NCHUNK - 2, (NCHUNK - 2) % 3, issue_g=False)
    step(NCHUNK - 1, (NCHUNK - 1) % 3, issue_g=False, issue_d=False)

    plsc.subcore_barrier()

    pltpu.sync_copy(
        acc_sh.at[pl.ds(s * RPT, RPT)],
        out_hbm.at[c, pl.ds(s * RPT, RPT)],
    )

    @pl.when(s == NS - 1)
    def _tail():
        pltpu.sync_copy(
            acc_sh.at[pl.ds(NS * RPT, N - NS * RPT)],
            out_hbm.at[c, pl.ds(NS * RPT, N - NS * RPT)],
        )


BR = 2000


def _mm_body(p_ref, w_ref, o_ref):
    a = p_ref[0] + p_ref[1]
    o_ref[...] = jnp.dot(a, w_ref[...], preferred_element_type=jnp.float32)


def _matmul(partials, W):
    return pl.pallas_call(
        _mm_body,
        grid=(N // BR,),
        in_specs=[
            pl.BlockSpec((NC, BR, D), lambda i: (0, i, 0)),
            pl.BlockSpec((D, D), lambda i: (0, 0)),
        ],
        out_specs=pl.BlockSpec((BR, D), lambda i: (i, 0)),
        out_shape=jax.ShapeDtypeStruct((N, D), jnp.float32),
    )(partials, W)


@jax.jit
def _run(x, edge_index, W):
    partials = _aggregate(x, edge_index.astype(jnp.int32).reshape(-1))
    return _matmul(partials, W)


def kernel(x, edge_index, W):
    return _run(x, edge_index, W)
